# baseline (speedup 1.0000x reference)
import functools

import jax
import jax.numpy as jnp
import numpy as np
from jax import lax
from jax.experimental import pallas as pl
from jax.experimental.pallas import tpu as pltpu
from jax.experimental.pallas import tpu_sc as plsc

N = 50000
E = 800000
G = 256
IN_SITE = 92
EXP = 41
SED = 64
BED = 32
H1 = 128
H2 = 64
MAXD = 8.0
STEP = MAXD / EXP

BN = 2000
NBLK = N // BN
BEB = 8000
EBLK = E // BEB

NSUB = 16
CB = 80
EPS = E // NSUB
NCHUNK = EPS // CB
NPS = 3128
NPS_LAST = N - (NSUB - 1) * NPS

F32 = jnp.float32
BF16 = jnp.bfloat16
INTER = plsc.PackFormat.INTERLEAVED

PERM64 = np.concatenate([
    np.arange(0, 16), np.arange(32, 48), np.arange(16, 32), np.arange(48, 64),
])


def _softplus(x):
    return jnp.maximum(x, 0.0) + jnp.log1p(jnp.exp(-jnp.abs(x)))


def _pack_pairs(x):
    b = lax.bitcast_convert_type(x, jnp.int32)
    odd = jnp.right_shift(b, 16) & 1
    bb = jnp.right_shift(b + 0x7FFF + odd, 16) & 0xFFFF
    lo, hi = bb[:, :32], bb[:, 32:]
    return lax.bitcast_convert_type(lo | (hi << 16), F32)


def _site_body(x_ref, ws_ref, bs_ref, wa_ref, wb_ref, s0_ref, ta_ref, tb_ref):
    x = x_ref[...]
    s0 = _softplus(jnp.dot(x, ws_ref[...], preferred_element_type=F32) + bs_ref[...])
    s0_ref[...] = s0
    for c in range(2):
        ta_ref[c] = _pack_pairs(jnp.dot(s0, wa_ref[c], preferred_element_type=F32))
        tb_ref[c] = _pack_pairs(jnp.dot(s0, wb_ref[c], preferred_element_type=F32))


def _site_embed(sites, W_site, b_site, WA, WB):
    return pl.pallas_call(
        _site_body,
        grid=(NBLK,),
        in_specs=[
            pl.BlockSpec((BN, IN_SITE), lambda i: (i, 0)),
            pl.BlockSpec((IN_SITE, SED), lambda i: (0, 0)),
            pl.BlockSpec((1, SED), lambda i: (0, 0)),
            pl.BlockSpec((2, SED, SED), lambda i: (0, 0, 0)),
            pl.BlockSpec((2, SED, SED), lambda i: (0, 0, 0)),
        ],
        out_specs=[
            pl.BlockSpec((BN, SED), lambda i: (i, 0)),
            pl.BlockSpec((2, BN, 32), lambda i: (0, i, 0)),
            pl.BlockSpec((2, BN, 32), lambda i: (0, i, 0)),
        ],
        out_shape=[
            jax.ShapeDtypeStruct((N, SED), F32),
            jax.ShapeDtypeStruct((2, N, 32), F32),
            jax.ShapeDtypeStruct((2, N, 32), F32),
        ],
    )(sites, W_site, b_site, WA, WB)


def _mid_body(s0_ref, m_ref, wa_ref, wb_ref, s1_ref, ta_ref, tb_ref):
    s1 = s0_ref[...] + jnp.concatenate([m_ref[0], m_ref[1]], axis=1)
    s1_ref[...] = s1
    for c in range(2):
        ta_ref[c] = _pack_pairs(jnp.dot(s1, wa_ref[c], preferred_element_type=F32))
        tb_ref[c] = _pack_pairs(jnp.dot(s1, wb_ref[c], preferred_element_type=F32))


def _mid_embed(s0, M, WA, WB):
    return pl.pallas_call(
        _mid_body,
        grid=(NBLK,),
        in_specs=[
            pl.BlockSpec((BN, SED), lambda i: (i, 0)),
            pl.BlockSpec((2, BN, 32), lambda i: (0, i, 0)),
            pl.BlockSpec((2, SED, SED), lambda i: (0, 0, 0)),
            pl.BlockSpec((2, SED, SED), lambda i: (0, 0, 0)),
        ],
        out_specs=[
            pl.BlockSpec((BN, SED), lambda i: (i, 0)),
            pl.BlockSpec((2, BN, 32), lambda i: (0, i, 0)),
            pl.BlockSpec((2, BN, 32), lambda i: (0, i, 0)),
        ],
        out_shape=[
            jax.ShapeDtypeStruct((N, SED), F32),
            jax.ShapeDtypeStruct((2, N, 32), F32),
            jax.ShapeDtypeStruct((2, N, 32), F32),
        ],
    )(s0, M, WA, WB)


def _bond_base_body(d_ref, wb_ref, bb_ref, be_ref):
    d = d_ref[...]
    f = lax.broadcasted_iota(jnp.int32, (BEB, EXP), 1).astype(F32) * STEP
    basis = jnp.exp(-((d - f) ** 2) / (STEP * STEP))
    be_ref[...] = _softplus(jnp.dot(basis, wb_ref[...], preferred_element_type=F32) + bb_ref[...])


def _bond_base(bonds, W_bond, b_bond):
    return pl.pallas_call(
        _bond_base_body,
        grid=(EBLK,),
        in_specs=[
            pl.BlockSpec((BEB, 1), lambda i: (i, 0)),
            pl.BlockSpec((EXP, BED), lambda i: (0, 0)),
            pl.BlockSpec((1, BED), lambda i: (0, 0)),
        ],
        out_specs=pl.BlockSpec((BEB, BED), lambda i: (i, 0)),
        out_shape=jax.ShapeDtypeStruct((E, BED), F32),
    )(bonds, W_bond, b_bond)


def _bond_term_body(be_ref, we_ref, eb_ref, o_ref):
    be = be_ref[...]
    for c in range(2):
        o_ref[c] = _pack_pairs(jnp.dot(be, we_ref[c], preferred_element_type=F32) + eb_ref[c])


def _bond_term(be, WE, eb):
    return pl.pallas_call(
        _bond_term_body,
        grid=(EBLK,),
        in_specs=[
            pl.BlockSpec((BEB, BED), lambda i: (i, 0)),
            pl.BlockSpec((2, BED, SED), lambda i: (0, 0, 0)),
            pl.BlockSpec((2, 1, SED), lambda i: (0, 0, 0)),
        ],
        out_specs=pl.BlockSpec((2, BEB, 32), lambda i: (0, i, 0)),
        out_shape=jax.ShapeDtypeStruct((2, E, 32), F32),
    )(be, WE, eb)


def _conv_sc(ta, tb, be, i1, i2, zeros):
    mesh = plsc.VectorSubcoreMesh(core_axis_name="c", subcore_axis_name="s")

    @functools.partial(
        pl.kernel,
        out_type=jax.ShapeDtypeStruct((2 * N, 32), F32),
        mesh=mesh,
        scratch_types=[
            pltpu.VMEM((2, CB), jnp.int32),
            pltpu.VMEM((2, CB), jnp.int32),
            pltpu.VMEM((2, CB), jnp.int32),
            pltpu.VMEM((2, CB, 32), F32),
            pltpu.VMEM((2, CB, 32), F32),
            pltpu.VMEM((2, CB, 32), F32),
            pltpu.VMEM((CB, 32), F32),
            pltpu.VMEM_SHARED((N, 32), F32),
            pltpu.SemaphoreType.DMA((2,)),
            pltpu.SemaphoreType.DMA((2,)),
            pltpu.SemaphoreType.DMA((2,)),
            pltpu.SemaphoreType.DMA((2,)),
        ],
        compiler_params=pltpu.CompilerParams(use_tc_tiling_on_sc=False,
                                             needs_layout_passes=False),
    )
    def conv(ta_h, tb_h, be_h, i1_h, i2_h, z_h, out_h,
             i1b, g1, g2, rA, rB, rE, msg, acc, isem, asem, bsem, esem):
        c = lax.axis_index("c")
        s = lax.axis_index("s")
        coff = c * N
        @pl.when(s < NSUB - 1)
        def _():
            pltpu.sync_copy(z_h.at[pl.ds(s * NPS, NPS)], acc.at[pl.ds(s * NPS, NPS)])

        @pl.when(s == NSUB - 1)
        def _():
            pltpu.sync_copy(z_h.at[pl.ds(s * NPS, NPS_LAST)],
                            acc.at[pl.ds(s * NPS, NPS_LAST)])

        plsc.subcore_barrier()

        ebase = s * EPS

        def adjust_and_fire(k, b):
            e0 = ebase + k * CB
            for j in range(CB // 16):
                sl = pl.ds(j * 16, 16)
                g1.at[b][sl] = i1b.at[b][sl] + coff
                g2.at[b][sl] = g2.at[b][sl] + coff
            pltpu.async_copy(ta_h.at[g1.at[b]], rA.at[b], asem.at[b])
            pltpu.async_copy(tb_h.at[g2.at[b]], rB.at[b], bsem.at[b])
            pltpu.async_copy(be_h.at[pl.ds(c * E + e0, CB)], rE.at[b], esem.at[b])

        def fire_idx(k, b):
            e0 = ebase + k * CB
            pltpu.async_copy(i1_h.at[pl.ds(e0, CB)], i1b.at[b], isem.at[b])
            pltpu.async_copy(i2_h.at[pl.ds(e0, CB)], g2.at[b], isem.at[b])

        def wait_idx(b):
            pltpu.make_async_copy(i1_h.at[pl.ds(0, CB)], i1b.at[b], isem.at[b]).wait()
            pltpu.make_async_copy(i2_h.at[pl.ds(0, CB)], g2.at[b], isem.at[b]).wait()

        def wait_gathers(b):
            pltpu.make_async_copy(ta_h.at[pl.ds(0, CB)], rA.at[b], asem.at[b]).wait()
            pltpu.make_async_copy(tb_h.at[pl.ds(0, CB)], rB.at[b], bsem.at[b]).wait()
            pltpu.make_async_copy(be_h.at[pl.ds(0, CB)], rE.at[b], esem.at[b]).wait()

        pltpu.sync_copy(i1_h.at[pl.ds(ebase, CB)], i1b.at[0])
        pltpu.sync_copy(i2_h.at[pl.ds(ebase, CB)], g2.at[0])
        adjust_and_fire(0, 0)
        fire_idx(1, 1)

        def chunk(k, carry):
            b = lax.rem(k, 2)
            nb = 1 - b

            @pl.when(k + 1 < NCHUNK)
            def _():
                wait_idx(nb)
                adjust_and_fire(k + 1, nb)

            wait_gathers(b)
            rAb, rBb, rEb = rA.at[b], rB.at[b], rE.at[b]

            def _up(ref, e, o):
                return plsc.unpack(plsc.bitcast(ref[e, pl.ds(o, 16)], BF16),
                                   format=INTER)

            @plsc.parallel_loop(0, CB, unroll=4)
            def _(e):
                a0A, a1A = _up(rAb, e, 0)
                a2A, a3A = _up(rAb, e, 16)
                a0B, a1B = _up(rBb, e, 0)
                a2B, a3B = _up(rBb, e, 16)
                a0E, a1E = _up(rEb, e, 0)
                a2E, a3E = _up(rEb, e, 16)
                a0 = a0A + a0B + a0E
                a1 = a1A + a1B + a1E
                a2 = a2A + a2B + a2E
                a3 = a3A + a3B + a3E
                sg0 = 1.0 / (1.0 + jnp.exp(-a0))
                sg1 = 1.0 / (1.0 + jnp.exp(-a1))
                msg[e, pl.ds(0, 16)] = sg0 * jnp.maximum(a2, 0.0)
                msg[e, pl.ds(16, 16)] = sg1 * jnp.maximum(a3, 0.0)

            pltpu.sync_copy(msg, acc.at[i1b.at[b]], add=True)

            @pl.when(k + 2 < NCHUNK)
            def _():
                fire_idx(k + 2, b)

            return carry

        lax.fori_loop(0, NCHUNK, chunk, 0)
        plsc.subcore_barrier()

        @pl.when(s < NSUB - 1)
        def _():
            pltpu.sync_copy(acc.at[pl.ds(s * NPS, NPS)],
                            out_h.at[pl.ds(coff + s * NPS, NPS)])

        @pl.when(s == NSUB - 1)
        def _():
            pltpu.sync_copy(acc.at[pl.ds(s * NPS, NPS_LAST)],
                            out_h.at[pl.ds(coff + s * NPS, NPS_LAST)])

    return conv(ta, tb, be, i1, i2, zeros)


def _pool_body(s1_ref, m_ref, ids_ref, w1_ref, b1_ref, w2_ref, b2_ref,
               w3_ref, b3_ref, out_ref, ssum, cnt):
    i = pl.program_id(0)

    @pl.when(i == 0)
    def _():
        ssum[...] = jnp.zeros_like(ssum)
        cnt[...] = jnp.zeros_like(cnt)

    s2 = s1_ref[...] + jnp.concatenate([m_ref[0], m_ref[1]], axis=1)
    ids = ids_ref[0]
    gi = lax.broadcasted_iota(jnp.int32, (G, BN), 0)
    onehot = jnp.where(gi == ids, 1.0, 0.0)
    ssum[...] += jnp.dot(onehot, s2, preferred_element_type=F32)
    cnt[...] += jnp.sum(onehot, axis=1, keepdims=True)

    @pl.when(i == NBLK - 1)
    def _():
        vec = ssum[...] / jnp.maximum(cnt[...], 1.0)
        h = jnp.maximum(jnp.dot(vec, w1_ref[...], preferred_element_type=F32) + b1_ref[...], 0.0)
        h = jnp.maximum(jnp.dot(h, w2_ref[...], preferred_element_type=F32) + b2_ref[...], 0.0)
        out_ref[...] = jnp.dot(h, w3_ref[...], preferred_element_type=F32) + b3_ref[...]


def _pool_mlp(s1, M, ids3, W1, b1, W2, b2, W3, b3):
    return pl.pallas_call(
        _pool_body,
        grid=(NBLK,),
        in_specs=[
            pl.BlockSpec((BN, SED), lambda i: (i, 0)),
            pl.BlockSpec((2, BN, 32), lambda i: (0, i, 0)),
            pl.BlockSpec((1, 1, BN), lambda i: (i, 0, 0)),
            pl.BlockSpec((SED, H1), lambda i: (0, 0)),
            pl.BlockSpec((1, H1), lambda i: (0, 0)),
            pl.BlockSpec((H1, H2), lambda i: (0, 0)),
            pl.BlockSpec((1, H2), lambda i: (0, 0)),
            pl.BlockSpec((H2, 1), lambda i: (0, 0)),
            pl.BlockSpec((1, 1), lambda i: (0, 0)),
        ],
        out_specs=pl.BlockSpec((G, 1), lambda i: (0, 0)),
        out_shape=jax.ShapeDtypeStruct((G, 1), F32),
        scratch_shapes=[
            pltpu.VMEM((G, SED), F32),
            pltpu.VMEM((G, 1), F32),
        ],
    )(s1, M, ids3, W1, b1, W2, b2, W3, b3)


def _split_weights(Wsig, Wsoft, bsig, bsoft):
    WA = jnp.stack([jnp.concatenate(
        [Wsig[0:SED, c * 32:(c + 1) * 32], Wsoft[0:SED, c * 32:(c + 1) * 32]],
        axis=1) for c in range(2)])
    WB = jnp.stack([jnp.concatenate(
        [Wsig[SED:2 * SED, c * 32:(c + 1) * 32], Wsoft[SED:2 * SED, c * 32:(c + 1) * 32]],
        axis=1) for c in range(2)])
    WE = jnp.stack([jnp.concatenate(
        [Wsig[2 * SED:, c * 32:(c + 1) * 32], Wsoft[2 * SED:, c * 32:(c + 1) * 32]],
        axis=1) for c in range(2)])
    bb = jnp.stack([jnp.concatenate(
        [bsig[c * 32:(c + 1) * 32], bsoft[c * 32:(c + 1) * 32]])
        for c in range(2)])[:, None, :]
    return WA[:, :, PERM64], WB[:, :, PERM64], WE[:, :, PERM64], bb[:, :, PERM64]


def kernel(sites, bonds, indices1, indices2, graph_to_sites, W_site, b_site,
           W_bond, b_bond, Wsig1, bsig1, Wsoft1, bsoft1, Wsig2, bsig2,
           Wsoft2, bsoft2, W1, b1, W2, b2, W3, b3):
    WA1, WB1, WE1, eb1 = _split_weights(Wsig1, Wsoft1, bsig1, bsoft1)
    WA2, WB2, WE2, eb2 = _split_weights(Wsig2, Wsoft2, bsig2, bsoft2)

    s0, TA1, TB1 = _site_embed(sites, W_site, b_site[None, :], WA1, WB1)
    be = _bond_base(bonds[:, None], W_bond, b_bond[None, :])
    BE1 = _bond_term(be, WE1, eb1)
    BE2 = _bond_term(be, WE2, eb2)

    zeros = jnp.zeros((N, 32), F32)
    M1 = _conv_sc(TA1.reshape(2 * N, 32), TB1.reshape(2 * N, 32),
                  BE1.reshape(2 * E, 32), indices1, indices2, zeros)
    s1, TA2, TB2 = _mid_embed(s0, M1.reshape(2, N, 32), WA2, WB2)
    M2 = _conv_sc(TA2.reshape(2 * N, 32), TB2.reshape(2 * N, 32),
                  BE2.reshape(2 * E, 32), indices1, indices2, zeros)

    ids3 = graph_to_sites.reshape(NBLK, 1, BN)
    return _pool_mlp(s1, M2.reshape(2, N, 32), ids3, W1, b1[None, :],
                     W2, b2[None, :], W3, b3[None, :])


# speedup vs baseline: 1.0240x; 1.0240x over previous
import functools

import jax
import jax.numpy as jnp
import numpy as np
from jax import lax
from jax.experimental import pallas as pl
from jax.experimental.pallas import tpu as pltpu
from jax.experimental.pallas import tpu_sc as plsc

N = 50000
E = 800000
G = 256
IN_SITE = 92
EXP = 41
SED = 64
BED = 32
H1 = 128
H2 = 64
MAXD = 8.0
STEP = MAXD / EXP

BN = 2000
NBLK = N // BN
BEB = 8000
EBLK = E // BEB

NSUB = 16
CB = 80
EPS = E // NSUB
NCHUNK = EPS // CB
NPS = 3128
NPS_LAST = N - (NSUB - 1) * NPS

F32 = jnp.float32
BF16 = jnp.bfloat16
INTER = plsc.PackFormat.INTERLEAVED

PERM64 = np.concatenate([
    np.arange(0, 16), np.arange(32, 48), np.arange(16, 32), np.arange(48, 64),
])


def _softplus(x):
    return jnp.maximum(x, 0.0) + jnp.log1p(jnp.exp(-jnp.abs(x)))


def _pack_pairs(x):
    b = lax.bitcast_convert_type(x, jnp.int32)
    odd = jnp.right_shift(b, 16) & 1
    bb = jnp.right_shift(b + 0x7FFF + odd, 16) & 0xFFFF
    lo, hi = bb[:, :32], bb[:, 32:]
    return lax.bitcast_convert_type(lo | (hi << 16), F32)


def _site_body(x_ref, ws_ref, bs_ref, wa_ref, wb_ref, s0_ref, ta_ref, tb_ref):
    x = x_ref[...]
    s0 = _softplus(jnp.dot(x, ws_ref[...], preferred_element_type=F32) + bs_ref[...])
    s0_ref[...] = s0
    for c in range(2):
        ta_ref[c] = _pack_pairs(jnp.dot(s0, wa_ref[c], preferred_element_type=F32))
        tb_ref[c] = _pack_pairs(jnp.dot(s0, wb_ref[c], preferred_element_type=F32))


def _site_embed(sites, W_site, b_site, WA, WB):
    return pl.pallas_call(
        _site_body,
        grid=(NBLK,),
        in_specs=[
            pl.BlockSpec((BN, IN_SITE), lambda i: (i, 0)),
            pl.BlockSpec((IN_SITE, SED), lambda i: (0, 0)),
            pl.BlockSpec((1, SED), lambda i: (0, 0)),
            pl.BlockSpec((2, SED, SED), lambda i: (0, 0, 0)),
            pl.BlockSpec((2, SED, SED), lambda i: (0, 0, 0)),
        ],
        out_specs=[
            pl.BlockSpec((BN, SED), lambda i: (i, 0)),
            pl.BlockSpec((2, BN, 32), lambda i: (0, i, 0)),
            pl.BlockSpec((2, BN, 32), lambda i: (0, i, 0)),
        ],
        out_shape=[
            jax.ShapeDtypeStruct((N, SED), F32),
            jax.ShapeDtypeStruct((2, N, 32), F32),
            jax.ShapeDtypeStruct((2, N, 32), F32),
        ],
    )(sites, W_site, b_site, WA, WB)


def _mid_body(s0_ref, m_ref, wa_ref, wb_ref, s1_ref, ta_ref, tb_ref):
    s1 = s0_ref[...] + jnp.concatenate([m_ref[0], m_ref[1]], axis=1)
    s1_ref[...] = s1
    for c in range(2):
        ta_ref[c] = _pack_pairs(jnp.dot(s1, wa_ref[c], preferred_element_type=F32))
        tb_ref[c] = _pack_pairs(jnp.dot(s1, wb_ref[c], preferred_element_type=F32))


def _mid_embed(s0, M, WA, WB):
    return pl.pallas_call(
        _mid_body,
        grid=(NBLK,),
        in_specs=[
            pl.BlockSpec((BN, SED), lambda i: (i, 0)),
            pl.BlockSpec((2, BN, 32), lambda i: (0, i, 0)),
            pl.BlockSpec((2, SED, SED), lambda i: (0, 0, 0)),
            pl.BlockSpec((2, SED, SED), lambda i: (0, 0, 0)),
        ],
        out_specs=[
            pl.BlockSpec((BN, SED), lambda i: (i, 0)),
            pl.BlockSpec((2, BN, 32), lambda i: (0, i, 0)),
            pl.BlockSpec((2, BN, 32), lambda i: (0, i, 0)),
        ],
        out_shape=[
            jax.ShapeDtypeStruct((N, SED), F32),
            jax.ShapeDtypeStruct((2, N, 32), F32),
            jax.ShapeDtypeStruct((2, N, 32), F32),
        ],
    )(s0, M, WA, WB)


def _bond_body(d_ref, wb_ref, bb_ref, we1_ref, eb1_ref, we2_ref, eb2_ref,
               o1_ref, o2_ref):
    d = d_ref[...]
    f = lax.broadcasted_iota(jnp.int32, (BEB, EXP), 1).astype(F32) * STEP
    basis = jnp.exp(-((d - f) ** 2) / (STEP * STEP))
    be = _softplus(jnp.dot(basis, wb_ref[...], preferred_element_type=F32) + bb_ref[...])
    for c in range(2):
        o1_ref[c] = _pack_pairs(jnp.dot(be, we1_ref[c], preferred_element_type=F32) + eb1_ref[c])
        o2_ref[c] = _pack_pairs(jnp.dot(be, we2_ref[c], preferred_element_type=F32) + eb2_ref[c])


def _bond_embed(bonds, W_bond, b_bond, WE1, eb1, WE2, eb2):
    return pl.pallas_call(
        _bond_body,
        grid=(EBLK,),
        in_specs=[
            pl.BlockSpec((BEB, 1), lambda i: (i, 0)),
            pl.BlockSpec((EXP, BED), lambda i: (0, 0)),
            pl.BlockSpec((1, BED), lambda i: (0, 0)),
            pl.BlockSpec((2, BED, SED), lambda i: (0, 0, 0)),
            pl.BlockSpec((2, 1, SED), lambda i: (0, 0, 0)),
            pl.BlockSpec((2, BED, SED), lambda i: (0, 0, 0)),
            pl.BlockSpec((2, 1, SED), lambda i: (0, 0, 0)),
        ],
        out_specs=[
            pl.BlockSpec((2, BEB, 32), lambda i: (0, i, 0)),
            pl.BlockSpec((2, BEB, 32), lambda i: (0, i, 0)),
        ],
        out_shape=[
            jax.ShapeDtypeStruct((2, E, 32), F32),
            jax.ShapeDtypeStruct((2, E, 32), F32),
        ],
    )(bonds, W_bond, b_bond, WE1, eb1, WE2, eb2)


def _conv_sc(ta, tb, be, i1, i2, zeros):
    mesh = plsc.VectorSubcoreMesh(core_axis_name="c", subcore_axis_name="s")

    @functools.partial(
        pl.kernel,
        out_type=jax.ShapeDtypeStruct((2 * N, 32), F32),
        mesh=mesh,
        scratch_types=[
            pltpu.VMEM((2, CB), jnp.int32),
            pltpu.VMEM((2, CB), jnp.int32),
            pltpu.VMEM((2, CB), jnp.int32),
            pltpu.VMEM((2, CB, 32), F32),
            pltpu.VMEM((2, CB, 32), F32),
            pltpu.VMEM((2, CB, 32), F32),
            pltpu.VMEM((CB, 32), F32),
            pltpu.VMEM_SHARED((N, 32), F32),
            pltpu.SemaphoreType.DMA((2,)),
            pltpu.SemaphoreType.DMA((2,)),
            pltpu.SemaphoreType.DMA((2,)),
            pltpu.SemaphoreType.DMA((2,)),
        ],
        compiler_params=pltpu.CompilerParams(use_tc_tiling_on_sc=False,
                                             needs_layout_passes=False),
    )
    def conv(ta_h, tb_h, be_h, i1_h, i2_h, z_h, out_h,
             i1b, g1, g2, rA, rB, rE, msg, acc, isem, asem, bsem, esem):
        c = lax.axis_index("c")
        s = lax.axis_index("s")
        coff = c * N
        @pl.when(s < NSUB - 1)
        def _():
            pltpu.sync_copy(z_h.at[pl.ds(s * NPS, NPS)], acc.at[pl.ds(s * NPS, NPS)])

        @pl.when(s == NSUB - 1)
        def _():
            pltpu.sync_copy(z_h.at[pl.ds(s * NPS, NPS_LAST)],
                            acc.at[pl.ds(s * NPS, NPS_LAST)])

        plsc.subcore_barrier()

        ebase = s * EPS

        def adjust_and_fire(k, b):
            e0 = ebase + k * CB
            for j in range(CB // 16):
                sl = pl.ds(j * 16, 16)
                g1.at[b][sl] = i1b.at[b][sl] + coff
                g2.at[b][sl] = g2.at[b][sl] + coff
            pltpu.async_copy(ta_h.at[g1.at[b]], rA.at[b], asem.at[b])
            pltpu.async_copy(tb_h.at[g2.at[b]], rB.at[b], bsem.at[b])
            pltpu.async_copy(be_h.at[pl.ds(c * E + e0, CB)], rE.at[b], esem.at[b])

        def fire_idx(k, b):
            e0 = ebase + k * CB
            pltpu.async_copy(i1_h.at[pl.ds(e0, CB)], i1b.at[b], isem.at[b])
            pltpu.async_copy(i2_h.at[pl.ds(e0, CB)], g2.at[b], isem.at[b])

        def wait_idx(b):
            pltpu.make_async_copy(i1_h.at[pl.ds(0, CB)], i1b.at[b], isem.at[b]).wait()
            pltpu.make_async_copy(i2_h.at[pl.ds(0, CB)], g2.at[b], isem.at[b]).wait()

        def wait_gathers(b):
            pltpu.make_async_copy(ta_h.at[pl.ds(0, CB)], rA.at[b], asem.at[b]).wait()
            pltpu.make_async_copy(tb_h.at[pl.ds(0, CB)], rB.at[b], bsem.at[b]).wait()
            pltpu.make_async_copy(be_h.at[pl.ds(0, CB)], rE.at[b], esem.at[b]).wait()

        pltpu.sync_copy(i1_h.at[pl.ds(ebase, CB)], i1b.at[0])
        pltpu.sync_copy(i2_h.at[pl.ds(ebase, CB)], g2.at[0])
        adjust_and_fire(0, 0)
        fire_idx(1, 1)

        def chunk(k, carry):
            b = lax.rem(k, 2)
            nb = 1 - b

            @pl.when(k + 1 < NCHUNK)
            def _():
                wait_idx(nb)
                adjust_and_fire(k + 1, nb)

            wait_gathers(b)
            rAb, rBb, rEb = rA.at[b], rB.at[b], rE.at[b]

            def _up(ref, e, o):
                return plsc.unpack(plsc.bitcast(ref[e, pl.ds(o, 16)], BF16),
                                   format=INTER)

            @plsc.parallel_loop(0, CB, unroll=8)
            def _(e):
                a0A, a1A = _up(rAb, e, 0)
                a2A, a3A = _up(rAb, e, 16)
                a0B, a1B = _up(rBb, e, 0)
                a2B, a3B = _up(rBb, e, 16)
                a0E, a1E = _up(rEb, e, 0)
                a2E, a3E = _up(rEb, e, 16)
                a0 = a0A + a0B + a0E
                a1 = a1A + a1B + a1E
                a2 = a2A + a2B + a2E
                a3 = a3A + a3B + a3E
                sg0 = 1.0 / (1.0 + jnp.exp(-a0))
                sg1 = 1.0 / (1.0 + jnp.exp(-a1))
                msg[e, pl.ds(0, 16)] = sg0 * jnp.maximum(a2, 0.0)
                msg[e, pl.ds(16, 16)] = sg1 * jnp.maximum(a3, 0.0)

            pltpu.sync_copy(msg, acc.at[i1b.at[b]], add=True)

            @pl.when(k + 2 < NCHUNK)
            def _():
                fire_idx(k + 2, b)

            return carry

        lax.fori_loop(0, NCHUNK, chunk, 0)
        plsc.subcore_barrier()

        @pl.when(s < NSUB - 1)
        def _():
            pltpu.sync_copy(acc.at[pl.ds(s * NPS, NPS)],
                            out_h.at[pl.ds(coff + s * NPS, NPS)])

        @pl.when(s == NSUB - 1)
        def _():
            pltpu.sync_copy(acc.at[pl.ds(s * NPS, NPS_LAST)],
                            out_h.at[pl.ds(coff + s * NPS, NPS_LAST)])

    return conv(ta, tb, be, i1, i2, zeros)


def _pool_body(s1_ref, m_ref, ids_ref, w1_ref, b1_ref, w2_ref, b2_ref,
               w3_ref, b3_ref, out_ref, ssum, cnt):
    i = pl.program_id(0)

    @pl.when(i == 0)
    def _():
        ssum[...] = jnp.zeros_like(ssum)
        cnt[...] = jnp.zeros_like(cnt)

    s2 = s1_ref[...] + jnp.concatenate([m_ref[0], m_ref[1]], axis=1)
    ids = ids_ref[0]
    gi = lax.broadcasted_iota(jnp.int32, (G, BN), 0)
    onehot = jnp.where(gi == ids, 1.0, 0.0)
    ssum[...] += jnp.dot(onehot, s2, preferred_element_type=F32)
    cnt[...] += jnp.sum(onehot, axis=1, keepdims=True)

    @pl.when(i == NBLK - 1)
    def _():
        vec = ssum[...] / jnp.maximum(cnt[...], 1.0)
        h = jnp.maximum(jnp.dot(vec, w1_ref[...], preferred_element_type=F32) + b1_ref[...], 0.0)
        h = jnp.maximum(jnp.dot(h, w2_ref[...], preferred_element_type=F32) + b2_ref[...], 0.0)
        out_ref[...] = jnp.dot(h, w3_ref[...], preferred_element_type=F32) + b3_ref[...]


def _pool_mlp(s1, M, ids3, W1, b1, W2, b2, W3, b3):
    return pl.pallas_call(
        _pool_body,
        grid=(NBLK,),
        in_specs=[
            pl.BlockSpec((BN, SED), lambda i: (i, 0)),
            pl.BlockSpec((2, BN, 32), lambda i: (0, i, 0)),
            pl.BlockSpec((1, 1, BN), lambda i: (i, 0, 0)),
            pl.BlockSpec((SED, H1), lambda i: (0, 0)),
            pl.BlockSpec((1, H1), lambda i: (0, 0)),
            pl.BlockSpec((H1, H2), lambda i: (0, 0)),
            pl.BlockSpec((1, H2), lambda i: (0, 0)),
            pl.BlockSpec((H2, 1), lambda i: (0, 0)),
            pl.BlockSpec((1, 1), lambda i: (0, 0)),
        ],
        out_specs=pl.BlockSpec((G, 1), lambda i: (0, 0)),
        out_shape=jax.ShapeDtypeStruct((G, 1), F32),
        scratch_shapes=[
            pltpu.VMEM((G, SED), F32),
            pltpu.VMEM((G, 1), F32),
        ],
    )(s1, M, ids3, W1, b1, W2, b2, W3, b3)


def _split_weights(Wsig, Wsoft, bsig, bsoft):
    WA = jnp.stack([jnp.concatenate(
        [Wsig[0:SED, c * 32:(c + 1) * 32], Wsoft[0:SED, c * 32:(c + 1) * 32]],
        axis=1) for c in range(2)])
    WB = jnp.stack([jnp.concatenate(
        [Wsig[SED:2 * SED, c * 32:(c + 1) * 32], Wsoft[SED:2 * SED, c * 32:(c + 1) * 32]],
        axis=1) for c in range(2)])
    WE = jnp.stack([jnp.concatenate(
        [Wsig[2 * SED:, c * 32:(c + 1) * 32], Wsoft[2 * SED:, c * 32:(c + 1) * 32]],
        axis=1) for c in range(2)])
    bb = jnp.stack([jnp.concatenate(
        [bsig[c * 32:(c + 1) * 32], bsoft[c * 32:(c + 1) * 32]])
        for c in range(2)])[:, None, :]
    return WA[:, :, PERM64], WB[:, :, PERM64], WE[:, :, PERM64], bb[:, :, PERM64]


def kernel(sites, bonds, indices1, indices2, graph_to_sites, W_site, b_site,
           W_bond, b_bond, Wsig1, bsig1, Wsoft1, bsoft1, Wsig2, bsig2,
           Wsoft2, bsoft2, W1, b1, W2, b2, W3, b3):
    WA1, WB1, WE1, eb1 = _split_weights(Wsig1, Wsoft1, bsig1, bsoft1)
    WA2, WB2, WE2, eb2 = _split_weights(Wsig2, Wsoft2, bsig2, bsoft2)

    s0, TA1, TB1 = _site_embed(sites, W_site, b_site[None, :], WA1, WB1)
    BE1, BE2 = _bond_embed(bonds[:, None], W_bond, b_bond[None, :], WE1, eb1, WE2, eb2)

    zeros = jnp.zeros((N, 32), F32)
    M1 = _conv_sc(TA1.reshape(2 * N, 32), TB1.reshape(2 * N, 32),
                  BE1.reshape(2 * E, 32), indices1, indices2, zeros)
    s1, TA2, TB2 = _mid_embed(s0, M1.reshape(2, N, 32), WA2, WB2)
    M2 = _conv_sc(TA2.reshape(2 * N, 32), TB2.reshape(2 * N, 32),
                  BE2.reshape(2 * E, 32), indices1, indices2, zeros)

    ids3 = graph_to_sites.reshape(NBLK, 1, BN)
    return _pool_mlp(s1, M2.reshape(2, N, 32), ids3, W1, b1[None, :],
                     W2, b2[None, :], W3, b3[None, :])


# speedup vs baseline: 1.0695x; 1.0445x over previous
import functools

import jax
import jax.numpy as jnp
import numpy as np
from jax import lax
from jax.experimental import pallas as pl
from jax.experimental.pallas import tpu as pltpu
from jax.experimental.pallas import tpu_sc as plsc

N = 50000
E = 800000
G = 256
IN_SITE = 92
EXP = 41
SED = 64
BED = 32
H1 = 128
H2 = 64
MAXD = 8.0
STEP = MAXD / EXP

BN = 2000
NBLK = N // BN
BEB = 8000
EBLK = E // BEB

NSUB = 16
CB = 80
EPS = E // NSUB
NCHUNK = EPS // CB
NPS = 3128
NPS_LAST = N - (NSUB - 1) * NPS

F32 = jnp.float32
BF16 = jnp.bfloat16
INTER = plsc.PackFormat.INTERLEAVED

PERM64 = np.concatenate([
    np.arange(0, 16), np.arange(32, 48), np.arange(16, 32), np.arange(48, 64),
])


def _softplus(x):
    return jnp.maximum(x, 0.0) + jnp.log1p(jnp.exp(-jnp.abs(x)))


def _pack_pairs(x):
    b = lax.bitcast_convert_type(x, jnp.int32)
    odd = jnp.right_shift(b, 16) & 1
    bb = jnp.right_shift(b + 0x7FFF + odd, 16) & 0xFFFF
    lo, hi = bb[:, :32], bb[:, 32:]
    return lax.bitcast_convert_type(lo | (hi << 16), F32)


def _site_body(x_ref, ws_ref, bs_ref, wa_ref, wb_ref, s0_ref, ta_ref, tb_ref):
    x = x_ref[...]
    s0 = _softplus(jnp.dot(x, ws_ref[...], preferred_element_type=F32) + bs_ref[...])
    s0_ref[...] = s0
    for c in range(2):
        ta_ref[c] = _pack_pairs(jnp.dot(s0, wa_ref[c], preferred_element_type=F32))
        tb_ref[c] = _pack_pairs(jnp.dot(s0, wb_ref[c], preferred_element_type=F32))


def _site_embed(sites, W_site, b_site, WA, WB):
    return pl.pallas_call(
        _site_body,
        grid=(NBLK,),
        in_specs=[
            pl.BlockSpec((BN, IN_SITE), lambda i: (i, 0)),
            pl.BlockSpec((IN_SITE, SED), lambda i: (0, 0)),
            pl.BlockSpec((1, SED), lambda i: (0, 0)),
            pl.BlockSpec((2, SED, SED), lambda i: (0, 0, 0)),
            pl.BlockSpec((2, SED, SED), lambda i: (0, 0, 0)),
        ],
        out_specs=[
            pl.BlockSpec((BN, SED), lambda i: (i, 0)),
            pl.BlockSpec((2, BN, 32), lambda i: (0, i, 0)),
            pl.BlockSpec((2, BN, 32), lambda i: (0, i, 0)),
        ],
        out_shape=[
            jax.ShapeDtypeStruct((N, SED), F32),
            jax.ShapeDtypeStruct((2, N, 32), F32),
            jax.ShapeDtypeStruct((2, N, 32), F32),
        ],
    )(sites, W_site, b_site, WA, WB)


def _mid_body(s0_ref, m_ref, wa_ref, wb_ref, s1_ref, ta_ref, tb_ref):
    s1 = s0_ref[...] + jnp.concatenate([m_ref[0], m_ref[1]], axis=1)
    s1_ref[...] = s1
    for c in range(2):
        ta_ref[c] = _pack_pairs(jnp.dot(s1, wa_ref[c], preferred_element_type=F32))
        tb_ref[c] = _pack_pairs(jnp.dot(s1, wb_ref[c], preferred_element_type=F32))


def _mid_embed(s0, M, WA, WB):
    return pl.pallas_call(
        _mid_body,
        grid=(NBLK,),
        in_specs=[
            pl.BlockSpec((BN, SED), lambda i: (i, 0)),
            pl.BlockSpec((2, BN, 32), lambda i: (0, i, 0)),
            pl.BlockSpec((2, SED, SED), lambda i: (0, 0, 0)),
            pl.BlockSpec((2, SED, SED), lambda i: (0, 0, 0)),
        ],
        out_specs=[
            pl.BlockSpec((BN, SED), lambda i: (i, 0)),
            pl.BlockSpec((2, BN, 32), lambda i: (0, i, 0)),
            pl.BlockSpec((2, BN, 32), lambda i: (0, i, 0)),
        ],
        out_shape=[
            jax.ShapeDtypeStruct((N, SED), F32),
            jax.ShapeDtypeStruct((2, N, 32), F32),
            jax.ShapeDtypeStruct((2, N, 32), F32),
        ],
    )(s0, M, WA, WB)


def _bond_body(d_ref, wb_ref, bb_ref, we1_ref, eb1_ref, we2_ref, eb2_ref,
               o1_ref, o2_ref):
    d = d_ref[...]
    f = lax.broadcasted_iota(jnp.int32, (BEB, EXP), 1).astype(F32) * STEP
    basis = jnp.exp(-((d - f) ** 2) / (STEP * STEP))
    be = _softplus(jnp.dot(basis, wb_ref[...], preferred_element_type=F32) + bb_ref[...])
    for c in range(2):
        o1_ref[c] = _pack_pairs(jnp.dot(be, we1_ref[c], preferred_element_type=F32) + eb1_ref[c])
        o2_ref[c] = _pack_pairs(jnp.dot(be, we2_ref[c], preferred_element_type=F32) + eb2_ref[c])


def _bond_embed(bonds, W_bond, b_bond, WE1, eb1, WE2, eb2):
    return pl.pallas_call(
        _bond_body,
        grid=(EBLK,),
        in_specs=[
            pl.BlockSpec((BEB, 1), lambda i: (i, 0)),
            pl.BlockSpec((EXP, BED), lambda i: (0, 0)),
            pl.BlockSpec((1, BED), lambda i: (0, 0)),
            pl.BlockSpec((2, BED, SED), lambda i: (0, 0, 0)),
            pl.BlockSpec((2, 1, SED), lambda i: (0, 0, 0)),
            pl.BlockSpec((2, BED, SED), lambda i: (0, 0, 0)),
            pl.BlockSpec((2, 1, SED), lambda i: (0, 0, 0)),
        ],
        out_specs=[
            pl.BlockSpec((2, BEB, 32), lambda i: (0, i, 0)),
            pl.BlockSpec((2, BEB, 32), lambda i: (0, i, 0)),
        ],
        out_shape=[
            jax.ShapeDtypeStruct((2, E, 32), F32),
            jax.ShapeDtypeStruct((2, E, 32), F32),
        ],
    )(bonds, W_bond, b_bond, WE1, eb1, WE2, eb2)


def _conv_sc(ta, tb, be, i1, i2, zeros):
    mesh = plsc.VectorSubcoreMesh(core_axis_name="c", subcore_axis_name="s")

    @functools.partial(
        pl.kernel,
        out_type=jax.ShapeDtypeStruct((2 * N, 32), F32),
        mesh=mesh,
        scratch_types=[
            pltpu.VMEM((3, CB), jnp.int32),
            pltpu.VMEM((3, CB), jnp.int32),
            pltpu.VMEM((3, CB), jnp.int32),
            pltpu.VMEM((2, CB, 32), F32),
            pltpu.VMEM((2, CB, 32), F32),
            pltpu.VMEM((2, CB, 32), F32),
            pltpu.VMEM((2, CB, 32), F32),
            pltpu.VMEM_SHARED((N, 32), F32),
            pltpu.SemaphoreType.DMA((3,)),
            pltpu.SemaphoreType.DMA((2,)),
            pltpu.SemaphoreType.DMA((2,)),
            pltpu.SemaphoreType.DMA((2,)),
            pltpu.SemaphoreType.DMA((2,)),
        ],
        compiler_params=pltpu.CompilerParams(use_tc_tiling_on_sc=False,
                                             needs_layout_passes=False),
    )
    def conv(ta_h, tb_h, be_h, i1_h, i2_h, z_h, out_h,
             i1b, g1, g2, rA, rB, rE, msg, acc, isem, asem, bsem, esem, ssem):
        c = lax.axis_index("c")
        s = lax.axis_index("s")
        coff = c * N
        @pl.when(s < NSUB - 1)
        def _():
            pltpu.sync_copy(z_h.at[pl.ds(s * NPS, NPS)], acc.at[pl.ds(s * NPS, NPS)])

        @pl.when(s == NSUB - 1)
        def _():
            pltpu.sync_copy(z_h.at[pl.ds(s * NPS, NPS_LAST)],
                            acc.at[pl.ds(s * NPS, NPS_LAST)])

        plsc.subcore_barrier()

        ebase = s * EPS

        def adjust_and_fire(k, b, s):
            e0 = ebase + k * CB
            for j in range(CB // 16):
                sl = pl.ds(j * 16, 16)
                g1.at[s][sl] = i1b.at[s][sl] + coff
                g2.at[s][sl] = g2.at[s][sl] + coff
            pltpu.async_copy(ta_h.at[g1.at[s]], rA.at[b], asem.at[b])
            pltpu.async_copy(tb_h.at[g2.at[s]], rB.at[b], bsem.at[b])
            pltpu.async_copy(be_h.at[pl.ds(c * E + e0, CB)], rE.at[b], esem.at[b])

        def fire_idx(k, s):
            e0 = ebase + k * CB
            pltpu.async_copy(i1_h.at[pl.ds(e0, CB)], i1b.at[s], isem.at[s])
            pltpu.async_copy(i2_h.at[pl.ds(e0, CB)], g2.at[s], isem.at[s])

        def wait_idx(s):
            pltpu.make_async_copy(i1_h.at[pl.ds(0, CB)], i1b.at[s], isem.at[s]).wait()
            pltpu.make_async_copy(i2_h.at[pl.ds(0, CB)], g2.at[s], isem.at[s]).wait()

        def wait_gathers(b):
            pltpu.make_async_copy(ta_h.at[pl.ds(0, CB)], rA.at[b], asem.at[b]).wait()
            pltpu.make_async_copy(tb_h.at[pl.ds(0, CB)], rB.at[b], bsem.at[b]).wait()
            pltpu.make_async_copy(be_h.at[pl.ds(0, CB)], rE.at[b], esem.at[b]).wait()

        def wait_scatter(b):
            pltpu.make_async_copy(msg.at[b], acc.at[pl.ds(0, CB)], ssem.at[b]).wait()

        pltpu.sync_copy(i1_h.at[pl.ds(ebase, CB)], i1b.at[0])
        pltpu.sync_copy(i2_h.at[pl.ds(ebase, CB)], g2.at[0])
        adjust_and_fire(0, 0, 0)
        fire_idx(1, 1)

        def chunk(k, carry):
            b = lax.rem(k, 2)
            nb = 1 - b
            s = lax.rem(k, 3)
            sn = lax.rem(k + 1, 3)
            sn2 = lax.rem(k + 2, 3)

            @pl.when(k + 1 < NCHUNK)
            def _():
                wait_idx(sn)
                adjust_and_fire(k + 1, nb, sn)

            wait_gathers(b)
            rAb, rBb, rEb, msgb = rA.at[b], rB.at[b], rE.at[b], msg.at[b]

            def _up(ref, e, o):
                return plsc.unpack(plsc.bitcast(ref[e, pl.ds(o, 16)], BF16),
                                   format=INTER)

            @plsc.parallel_loop(0, CB, unroll=8)
            def _(e):
                a0A, a1A = _up(rAb, e, 0)
                a2A, a3A = _up(rAb, e, 16)
                a0B, a1B = _up(rBb, e, 0)
                a2B, a3B = _up(rBb, e, 16)
                a0E, a1E = _up(rEb, e, 0)
                a2E, a3E = _up(rEb, e, 16)
                a0 = a0A + a0B + a0E
                a1 = a1A + a1B + a1E
                a2 = a2A + a2B + a2E
                a3 = a3A + a3B + a3E
                sg0 = 1.0 / (1.0 + jnp.exp(-a0))
                sg1 = 1.0 / (1.0 + jnp.exp(-a1))
                msgb[e, pl.ds(0, 16)] = sg0 * jnp.maximum(a2, 0.0)
                msgb[e, pl.ds(16, 16)] = sg1 * jnp.maximum(a3, 0.0)

            pltpu.async_copy(msg.at[b], acc.at[i1b.at[s]], ssem.at[b], add=True)

            @pl.when(k >= 1)
            def _():
                wait_scatter(nb)

            @pl.when(k + 2 < NCHUNK)
            def _():
                fire_idx(k + 2, sn2)

            return carry

        lax.fori_loop(0, NCHUNK, chunk, 0)
        wait_scatter(lax.rem(NCHUNK - 1, 2))
        plsc.subcore_barrier()

        @pl.when(s < NSUB - 1)
        def _():
            pltpu.sync_copy(acc.at[pl.ds(s * NPS, NPS)],
                            out_h.at[pl.ds(coff + s * NPS, NPS)])

        @pl.when(s == NSUB - 1)
        def _():
            pltpu.sync_copy(acc.at[pl.ds(s * NPS, NPS_LAST)],
                            out_h.at[pl.ds(coff + s * NPS, NPS_LAST)])

    return conv(ta, tb, be, i1, i2, zeros)


def _pool_body(s1_ref, m_ref, ids_ref, w1_ref, b1_ref, w2_ref, b2_ref,
               w3_ref, b3_ref, out_ref, ssum, cnt):
    i = pl.program_id(0)

    @pl.when(i == 0)
    def _():
        ssum[...] = jnp.zeros_like(ssum)
        cnt[...] = jnp.zeros_like(cnt)

    s2 = s1_ref[...] + jnp.concatenate([m_ref[0], m_ref[1]], axis=1)
    ids = ids_ref[0]
    gi = lax.broadcasted_iota(jnp.int32, (G, BN), 0)
    onehot = jnp.where(gi == ids, 1.0, 0.0)
    ssum[...] += jnp.dot(onehot, s2, preferred_element_type=F32)
    cnt[...] += jnp.sum(onehot, axis=1, keepdims=True)

    @pl.when(i == NBLK - 1)
    def _():
        vec = ssum[...] / jnp.maximum(cnt[...], 1.0)
        h = jnp.maximum(jnp.dot(vec, w1_ref[...], preferred_element_type=F32) + b1_ref[...], 0.0)
        h = jnp.maximum(jnp.dot(h, w2_ref[...], preferred_element_type=F32) + b2_ref[...], 0.0)
        out_ref[...] = jnp.dot(h, w3_ref[...], preferred_element_type=F32) + b3_ref[...]


def _pool_mlp(s1, M, ids3, W1, b1, W2, b2, W3, b3):
    return pl.pallas_call(
        _pool_body,
        grid=(NBLK,),
        in_specs=[
            pl.BlockSpec((BN, SED), lambda i: (i, 0)),
            pl.BlockSpec((2, BN, 32), lambda i: (0, i, 0)),
            pl.BlockSpec((1, 1, BN), lambda i: (i, 0, 0)),
            pl.BlockSpec((SED, H1), lambda i: (0, 0)),
            pl.BlockSpec((1, H1), lambda i: (0, 0)),
            pl.BlockSpec((H1, H2), lambda i: (0, 0)),
            pl.BlockSpec((1, H2), lambda i: (0, 0)),
            pl.BlockSpec((H2, 1), lambda i: (0, 0)),
            pl.BlockSpec((1, 1), lambda i: (0, 0)),
        ],
        out_specs=pl.BlockSpec((G, 1), lambda i: (0, 0)),
        out_shape=jax.ShapeDtypeStruct((G, 1), F32),
        scratch_shapes=[
            pltpu.VMEM((G, SED), F32),
            pltpu.VMEM((G, 1), F32),
        ],
    )(s1, M, ids3, W1, b1, W2, b2, W3, b3)


def _split_weights(Wsig, Wsoft, bsig, bsoft):
    WA = jnp.stack([jnp.concatenate(
        [Wsig[0:SED, c * 32:(c + 1) * 32], Wsoft[0:SED, c * 32:(c + 1) * 32]],
        axis=1) for c in range(2)])
    WB = jnp.stack([jnp.concatenate(
        [Wsig[SED:2 * SED, c * 32:(c + 1) * 32], Wsoft[SED:2 * SED, c * 32:(c + 1) * 32]],
        axis=1) for c in range(2)])
    WE = jnp.stack([jnp.concatenate(
        [Wsig[2 * SED:, c * 32:(c + 1) * 32], Wsoft[2 * SED:, c * 32:(c + 1) * 32]],
        axis=1) for c in range(2)])
    bb = jnp.stack([jnp.concatenate(
        [bsig[c * 32:(c + 1) * 32], bsoft[c * 32:(c + 1) * 32]])
        for c in range(2)])[:, None, :]
    return WA[:, :, PERM64], WB[:, :, PERM64], WE[:, :, PERM64], bb[:, :, PERM64]


def kernel(sites, bonds, indices1, indices2, graph_to_sites, W_site, b_site,
           W_bond, b_bond, Wsig1, bsig1, Wsoft1, bsoft1, Wsig2, bsig2,
           Wsoft2, bsoft2, W1, b1, W2, b2, W3, b3):
    WA1, WB1, WE1, eb1 = _split_weights(Wsig1, Wsoft1, bsig1, bsoft1)
    WA2, WB2, WE2, eb2 = _split_weights(Wsig2, Wsoft2, bsig2, bsoft2)

    s0, TA1, TB1 = _site_embed(sites, W_site, b_site[None, :], WA1, WB1)
    BE1, BE2 = _bond_embed(bonds[:, None], W_bond, b_bond[None, :], WE1, eb1, WE2, eb2)

    zeros = jnp.zeros((N, 32), F32)
    M1 = _conv_sc(TA1.reshape(2 * N, 32), TB1.reshape(2 * N, 32),
                  BE1.reshape(2 * E, 32), indices1, indices2, zeros)
    s1, TA2, TB2 = _mid_embed(s0, M1.reshape(2, N, 32), WA2, WB2)
    M2 = _conv_sc(TA2.reshape(2 * N, 32), TB2.reshape(2 * N, 32),
                  BE2.reshape(2 * E, 32), indices1, indices2, zeros)

    ids3 = graph_to_sites.reshape(NBLK, 1, BN)
    return _pool_mlp(s1, M2.reshape(2, N, 32), ids3, W1, b1[None, :],
                     W2, b2[None, :], W3, b3[None, :])
